# Initial kernel scaffold; baseline (speedup 1.0000x reference)
import jax
import jax.numpy as jnp
from jax.experimental import pallas as pl


def kernel(x, row_embed, col_embed, time_embed, cls_token_pos):
    raise NotImplementedError("write your pallas kernel here")


# speedup vs baseline: 1.1039x; 1.1039x over previous
import jax
import jax.numpy as jnp
from jax.experimental import pallas as pl

GRID_H, GRID_W, EMBED_DIM = 32, 32, 768
D = EMBED_DIM // 3
N = GRID_H * GRID_W


def _pos_emb_kernel(row_ref, col_ref, time_ref, cls_ref, out_ref):
    row_grid = jnp.broadcast_to(row_ref[...][:, None, :], (GRID_H, GRID_W, D))
    col_grid = jnp.broadcast_to(col_ref[...][None, :, :], (GRID_H, GRID_W, D))
    out_ref[pl.ds(1, N), 0:D] = row_grid.reshape(N, D)
    out_ref[pl.ds(1, N), D:2 * D] = col_grid.reshape(N, D)
    out_ref[pl.ds(1, N), 2 * D:3 * D] = time_ref[...]
    out_ref[0:1, :] = cls_ref[0]


def kernel(x, row_embed, col_embed, time_embed, cls_token_pos):
    out = pl.pallas_call(
        _pos_emb_kernel,
        out_shape=jax.ShapeDtypeStruct((N + 1, EMBED_DIM), jnp.float32),
    )(row_embed, col_embed, time_embed, cls_token_pos)
    return out[None]
